# baseline (speedup 1.0000x reference)
import functools

import jax
import jax.numpy as jnp
from jax import lax
from jax.experimental import pallas as pl
from jax.experimental.pallas import tpu as pltpu
from jax.experimental.pallas import tpu_sc as plsc

_B = 256
_D = 4 * 64 * 64
_T_PAD = 1024
_LANES = 16


def _sc_gather_coeffs(t, sac_p, s1mac_p):
    info = plsc.get_sparse_core_info()
    num_cores = info.num_cores
    mesh = plsc.VectorSubcoreMesh(core_axis_name="c", subcore_axis_name="s")
    n_workers = _B // _LANES

    @functools.partial(
        pl.kernel,
        mesh=mesh,
        out_type=(
            jax.ShapeDtypeStruct((_B,), jnp.float32),
            jax.ShapeDtypeStruct((_B,), jnp.float32),
        ),
        scratch_types=[
            pltpu.VMEM((_LANES,), jnp.int32),
            pltpu.VMEM((_LANES,), jnp.float32),
            pltpu.VMEM((_LANES,), jnp.float32),
            pltpu.SemaphoreType.DMA,
            pltpu.SemaphoreType.DMA,
        ],
    )
    def gather_kernel(t_hbm, sac_hbm, s1mac_hbm, c1_hbm, c2_hbm,
                      idx_v, o1_v, o2_v, sem1, sem2):
        wid = lax.axis_index("s") * num_cores + lax.axis_index("c")

        @pl.when(wid < n_workers)
        def _():
            base = wid * _LANES
            pltpu.sync_copy(t_hbm.at[pl.ds(base, _LANES)], idx_v)
            cp1 = pltpu.async_copy(sac_hbm.at[idx_v], o1_v, sem1)
            cp2 = pltpu.async_copy(s1mac_hbm.at[idx_v], o2_v, sem2)
            cp1.wait()
            cp2.wait()
            pltpu.sync_copy(o1_v, c1_hbm.at[pl.ds(base, _LANES)])
            pltpu.sync_copy(o2_v, c2_hbm.at[pl.ds(base, _LANES)])

    return gather_kernel(t, sac_p, s1mac_p)


def _tc_combine(x2, n2, c1, c2):
    bb = 16
    grid = (_B // bb,)

    def body(c1_ref, c2_ref, x_ref, n_ref, o_ref):
        o_ref[...] = c1_ref[...] * x_ref[...] + c2_ref[...] * n_ref[...]

    return pl.pallas_call(
        body,
        grid=grid,
        in_specs=[
            pl.BlockSpec((bb, 1), lambda i: (i, 0)),
            pl.BlockSpec((bb, 1), lambda i: (i, 0)),
            pl.BlockSpec((bb, _D), lambda i: (i, 0)),
            pl.BlockSpec((bb, _D), lambda i: (i, 0)),
        ],
        out_specs=pl.BlockSpec((bb, _D), lambda i: (i, 0)),
        out_shape=jax.ShapeDtypeStruct((_B, _D), jnp.float32),
        compiler_params=pltpu.CompilerParams(
            dimension_semantics=("arbitrary",)),
    )(c1, c2, x2, n2)


def kernel(x_start, t, noise, sqrt_alphas_cumprod, sqrt_one_minus_alphas_cumprod):
    B, C, H, W = x_start.shape
    sac_p = jnp.pad(sqrt_alphas_cumprod, (0, _T_PAD - sqrt_alphas_cumprod.shape[0]))
    s1mac_p = jnp.pad(sqrt_one_minus_alphas_cumprod,
                      (0, _T_PAD - sqrt_one_minus_alphas_cumprod.shape[0]))
    c1 = jnp.ones((B,), jnp.float32)
    c2 = jnp.ones((B,), jnp.float32)
    x2 = x_start.reshape(B, C * H * W)
    n2 = noise.reshape(B, C * H * W)
    out = _tc_combine(x2, n2, c1.reshape(B, 1), c2.reshape(B, 1))
    return out.reshape(B, C, H, W)


# speedup vs baseline: 1.0370x; 1.0370x over previous
import functools

import jax
import jax.numpy as jnp
from jax import lax
from jax.experimental import pallas as pl
from jax.experimental.pallas import tpu as pltpu
from jax.experimental.pallas import tpu_sc as plsc

_B = 256
_D = 4 * 64 * 64
_T_PAD = 1024
_LANES = 16


def _sc_gather_coeffs(t, sac_p, s1mac_p):
    info = plsc.get_sparse_core_info()
    num_cores = info.num_cores
    mesh = plsc.VectorSubcoreMesh(core_axis_name="c", subcore_axis_name="s")
    n_workers = _B // _LANES

    @functools.partial(
        pl.kernel,
        mesh=mesh,
        out_type=(
            jax.ShapeDtypeStruct((_B,), jnp.float32),
            jax.ShapeDtypeStruct((_B,), jnp.float32),
        ),
        scratch_types=[
            pltpu.VMEM((_LANES,), jnp.int32),
            pltpu.VMEM((_LANES,), jnp.float32),
            pltpu.VMEM((_LANES,), jnp.float32),
            pltpu.SemaphoreType.DMA,
            pltpu.SemaphoreType.DMA,
        ],
    )
    def gather_kernel(t_hbm, sac_hbm, s1mac_hbm, c1_hbm, c2_hbm,
                      idx_v, o1_v, o2_v, sem1, sem2):
        wid = lax.axis_index("s") * num_cores + lax.axis_index("c")

        @pl.when(wid < n_workers)
        def _():
            base = wid * _LANES
            pltpu.sync_copy(t_hbm.at[pl.ds(base, _LANES)], idx_v)
            cp1 = pltpu.async_copy(sac_hbm.at[idx_v], o1_v, sem1)
            cp2 = pltpu.async_copy(s1mac_hbm.at[idx_v], o2_v, sem2)
            cp1.wait()
            cp2.wait()
            pltpu.sync_copy(o1_v, c1_hbm.at[pl.ds(base, _LANES)])
            pltpu.sync_copy(o2_v, c2_hbm.at[pl.ds(base, _LANES)])

    return gather_kernel(t, sac_p, s1mac_p)


_STEPS = 8
_BB = _B // _STEPS
_NQ = 4
_RQ = _BB // _NQ


def _tc_combine(x2, n2, c1, c2):
    def body(c1_ref, c2_ref, x_hbm, n_hbm, o_hbm, xb, nb, ob, sx, sn, so):
        def fetch(s):
            slot = s % 2
            for q in range(_NQ):
                r = s * _BB + q * _RQ
                pltpu.make_async_copy(
                    x_hbm.at[pl.ds(r, _RQ), :],
                    xb.at[slot, pl.ds(q * _RQ, _RQ), :],
                    sx.at[slot, q]).start()
                pltpu.make_async_copy(
                    n_hbm.at[pl.ds(r, _RQ), :],
                    nb.at[slot, pl.ds(q * _RQ, _RQ), :],
                    sn.at[slot, q]).start()

        def wait_fetch(s):
            slot = s % 2
            for q in range(_NQ):
                r = s * _BB + q * _RQ
                pltpu.make_async_copy(
                    x_hbm.at[pl.ds(r, _RQ), :],
                    xb.at[slot, pl.ds(q * _RQ, _RQ), :],
                    sx.at[slot, q]).wait()
                pltpu.make_async_copy(
                    n_hbm.at[pl.ds(r, _RQ), :],
                    nb.at[slot, pl.ds(q * _RQ, _RQ), :],
                    sn.at[slot, q]).wait()

        def store(s):
            slot = s % 2
            for q in range(_NQ):
                r = s * _BB + q * _RQ
                pltpu.make_async_copy(
                    ob.at[slot, pl.ds(q * _RQ, _RQ), :],
                    o_hbm.at[pl.ds(r, _RQ), :],
                    so.at[slot, q]).start()

        def wait_store(s):
            slot = s % 2
            for q in range(_NQ):
                r = s * _BB + q * _RQ
                pltpu.make_async_copy(
                    ob.at[slot, pl.ds(q * _RQ, _RQ), :],
                    o_hbm.at[pl.ds(r, _RQ), :],
                    so.at[slot, q]).wait()

        fetch(0)
        for s in range(_STEPS):
            slot = s % 2
            if s + 1 < _STEPS:
                fetch(s + 1)
            wait_fetch(s)
            if s >= 2:
                wait_store(s - 2)
            cc1 = c1_ref[pl.ds(s * _BB, _BB), :]
            cc2 = c2_ref[pl.ds(s * _BB, _BB), :]
            ob[slot] = cc1 * xb[slot] + cc2 * nb[slot]
            store(s)
        wait_store(_STEPS - 2)
        wait_store(_STEPS - 1)

    return pl.pallas_call(
        body,
        in_specs=[
            pl.BlockSpec(memory_space=pltpu.MemorySpace.VMEM),
            pl.BlockSpec(memory_space=pltpu.MemorySpace.VMEM),
            pl.BlockSpec(memory_space=pl.ANY),
            pl.BlockSpec(memory_space=pl.ANY),
        ],
        out_specs=pl.BlockSpec(memory_space=pl.ANY),
        out_shape=jax.ShapeDtypeStruct((_B, _D), jnp.float32),
        scratch_shapes=[
            pltpu.VMEM((2, _BB, _D), jnp.float32),
            pltpu.VMEM((2, _BB, _D), jnp.float32),
            pltpu.VMEM((2, _BB, _D), jnp.float32),
            pltpu.SemaphoreType.DMA((2, _NQ)),
            pltpu.SemaphoreType.DMA((2, _NQ)),
            pltpu.SemaphoreType.DMA((2, _NQ)),
        ],
    )(c1, c2, x2, n2)


def kernel(x_start, t, noise, sqrt_alphas_cumprod, sqrt_one_minus_alphas_cumprod):
    B, C, H, W = x_start.shape
    sac_p = jnp.pad(sqrt_alphas_cumprod, (0, _T_PAD - sqrt_alphas_cumprod.shape[0]))
    s1mac_p = jnp.pad(sqrt_one_minus_alphas_cumprod,
                      (0, _T_PAD - sqrt_one_minus_alphas_cumprod.shape[0]))
    c1 = jnp.ones((B,), jnp.float32)
    c2 = jnp.ones((B,), jnp.float32)
    x2 = x_start.reshape(B, C * H * W)
    n2 = noise.reshape(B, C * H * W)
    out = _tc_combine(x2, n2, c1.reshape(B, 1), c2.reshape(B, 1))
    return out.reshape(B, C, H, W)


# speedup vs baseline: 1.0485x; 1.0111x over previous
import functools

import jax
import jax.numpy as jnp
from jax import lax
from jax.experimental import pallas as pl
from jax.experimental.pallas import tpu as pltpu
from jax.experimental.pallas import tpu_sc as plsc

_B = 256
_D = 4 * 64 * 64
_T_PAD = 1024
_LANES = 16


def _sc_gather_coeffs(t, sac_p, s1mac_p):
    info = plsc.get_sparse_core_info()
    num_cores = info.num_cores
    mesh = plsc.VectorSubcoreMesh(core_axis_name="c", subcore_axis_name="s")
    n_workers = _B // _LANES

    @functools.partial(
        pl.kernel,
        mesh=mesh,
        out_type=(
            jax.ShapeDtypeStruct((_B,), jnp.float32),
            jax.ShapeDtypeStruct((_B,), jnp.float32),
        ),
        scratch_types=[
            pltpu.VMEM((_LANES,), jnp.int32),
            pltpu.VMEM((_LANES,), jnp.float32),
            pltpu.VMEM((_LANES,), jnp.float32),
            pltpu.SemaphoreType.DMA,
            pltpu.SemaphoreType.DMA,
        ],
    )
    def gather_kernel(t_hbm, sac_hbm, s1mac_hbm, c1_hbm, c2_hbm,
                      idx_v, o1_v, o2_v, sem1, sem2):
        wid = lax.axis_index("s") * num_cores + lax.axis_index("c")

        @pl.when(wid < n_workers)
        def _():
            base = wid * _LANES
            pltpu.sync_copy(t_hbm.at[pl.ds(base, _LANES)], idx_v)
            cp1 = pltpu.async_copy(sac_hbm.at[idx_v], o1_v, sem1)
            cp2 = pltpu.async_copy(s1mac_hbm.at[idx_v], o2_v, sem2)
            cp1.wait()
            cp2.wait()
            pltpu.sync_copy(o1_v, c1_hbm.at[pl.ds(base, _LANES)])
            pltpu.sync_copy(o2_v, c2_hbm.at[pl.ds(base, _LANES)])

    return gather_kernel(t, sac_p, s1mac_p)


_STEPS = 4
_BB = _B // _STEPS
_NQ = 1
_RQ = _BB // _NQ


def _tc_combine(x2, n2, c1, c2):
    def body(c1_ref, c2_ref, x_hbm, n_hbm, o_hbm, xb, nb, ob, sx, sn, so):
        def fetch(s):
            slot = s % 2
            for q in range(_NQ):
                r = s * _BB + q * _RQ
                pltpu.make_async_copy(
                    x_hbm.at[pl.ds(r, _RQ), :],
                    xb.at[slot, pl.ds(q * _RQ, _RQ), :],
                    sx.at[slot, q]).start()
                pltpu.make_async_copy(
                    n_hbm.at[pl.ds(r, _RQ), :],
                    nb.at[slot, pl.ds(q * _RQ, _RQ), :],
                    sn.at[slot, q]).start()

        def wait_fetch(s):
            slot = s % 2
            for q in range(_NQ):
                r = s * _BB + q * _RQ
                pltpu.make_async_copy(
                    x_hbm.at[pl.ds(r, _RQ), :],
                    xb.at[slot, pl.ds(q * _RQ, _RQ), :],
                    sx.at[slot, q]).wait()
                pltpu.make_async_copy(
                    n_hbm.at[pl.ds(r, _RQ), :],
                    nb.at[slot, pl.ds(q * _RQ, _RQ), :],
                    sn.at[slot, q]).wait()

        def store(s):
            slot = s % 2
            for q in range(_NQ):
                r = s * _BB + q * _RQ
                pltpu.make_async_copy(
                    ob.at[slot, pl.ds(q * _RQ, _RQ), :],
                    o_hbm.at[pl.ds(r, _RQ), :],
                    so.at[slot, q]).start()

        def wait_store(s):
            slot = s % 2
            for q in range(_NQ):
                r = s * _BB + q * _RQ
                pltpu.make_async_copy(
                    ob.at[slot, pl.ds(q * _RQ, _RQ), :],
                    o_hbm.at[pl.ds(r, _RQ), :],
                    so.at[slot, q]).wait()

        fetch(0)
        for s in range(_STEPS):
            slot = s % 2
            if s + 1 < _STEPS:
                fetch(s + 1)
            wait_fetch(s)
            if s >= 2:
                wait_store(s - 2)
            cc1 = c1_ref[pl.ds(s * _BB, _BB), :]
            cc2 = c2_ref[pl.ds(s * _BB, _BB), :]
            ob[slot] = cc1 * xb[slot] + cc2 * nb[slot]
            store(s)
        wait_store(_STEPS - 2)
        wait_store(_STEPS - 1)

    return pl.pallas_call(
        body,
        in_specs=[
            pl.BlockSpec(memory_space=pltpu.MemorySpace.VMEM),
            pl.BlockSpec(memory_space=pltpu.MemorySpace.VMEM),
            pl.BlockSpec(memory_space=pl.ANY),
            pl.BlockSpec(memory_space=pl.ANY),
        ],
        out_specs=pl.BlockSpec(memory_space=pl.ANY),
        out_shape=jax.ShapeDtypeStruct((_B, _D), jnp.float32),
        scratch_shapes=[
            pltpu.VMEM((2, _BB, _D), jnp.float32),
            pltpu.VMEM((2, _BB, _D), jnp.float32),
            pltpu.VMEM((2, _BB, _D), jnp.float32),
            pltpu.SemaphoreType.DMA((2, _NQ)),
            pltpu.SemaphoreType.DMA((2, _NQ)),
            pltpu.SemaphoreType.DMA((2, _NQ)),
        ],
    )(c1, c2, x2, n2)


def kernel(x_start, t, noise, sqrt_alphas_cumprod, sqrt_one_minus_alphas_cumprod):
    B, C, H, W = x_start.shape
    sac_p = jnp.pad(sqrt_alphas_cumprod, (0, _T_PAD - sqrt_alphas_cumprod.shape[0]))
    s1mac_p = jnp.pad(sqrt_one_minus_alphas_cumprod,
                      (0, _T_PAD - sqrt_one_minus_alphas_cumprod.shape[0]))
    c1 = jnp.ones((B,), jnp.float32)
    c2 = jnp.ones((B,), jnp.float32)
    x2 = x_start.reshape(B, C * H * W)
    n2 = noise.reshape(B, C * H * W)
    out = _tc_combine(x2, n2, c1.reshape(B, 1), c2.reshape(B, 1))
    return out.reshape(B, C, H, W)


# speedup vs baseline: 1.8794x; 1.7924x over previous
import functools

import jax
import jax.numpy as jnp
from jax import lax
from jax.experimental import pallas as pl
from jax.experimental.pallas import tpu as pltpu
from jax.experimental.pallas import tpu_sc as plsc

_B = 256
_D = 4 * 64 * 64
_CHUNK = 128


def _sc_gather_coeffs(t, sac, s1mac):
    info = plsc.get_sparse_core_info()
    num_cores = info.num_cores
    mesh = plsc.VectorSubcoreMesh(core_axis_name="c", subcore_axis_name="s")
    n_workers = _B // _CHUNK

    @functools.partial(
        pl.kernel,
        mesh=mesh,
        out_type=(
            jax.ShapeDtypeStruct((_B,), jnp.float32),
            jax.ShapeDtypeStruct((_B,), jnp.float32),
        ),
        scratch_types=[
            pltpu.VMEM((_CHUNK,), jnp.int32),
            pltpu.VMEM((_CHUNK,), jnp.float32),
            pltpu.VMEM((_CHUNK,), jnp.float32),
            pltpu.SemaphoreType.DMA,
            pltpu.SemaphoreType.DMA,
        ],
    )
    def gather_kernel(t_hbm, sac_hbm, s1mac_hbm, c1_hbm, c2_hbm,
                      idx_v, o1_v, o2_v, sem1, sem2):
        wid = lax.axis_index("s") * num_cores + lax.axis_index("c")

        @pl.when(wid < n_workers)
        def _():
            base = wid * _CHUNK
            pltpu.sync_copy(t_hbm.at[pl.ds(base, _CHUNK)], idx_v)
            cp1 = pltpu.async_copy(sac_hbm.at[idx_v], o1_v, sem1)
            cp2 = pltpu.async_copy(s1mac_hbm.at[idx_v], o2_v, sem2)
            cp1.wait()
            cp2.wait()
            pltpu.sync_copy(o1_v, c1_hbm.at[pl.ds(base, _CHUNK)])
            pltpu.sync_copy(o2_v, c2_hbm.at[pl.ds(base, _CHUNK)])

    return gather_kernel(t, sac, s1mac)


def _tc_combine(xt, nt, c1r, c2r):
    bf = 2048
    grid = (_D // bf,)

    def body(c1_ref, c2_ref, x_ref, n_ref, o_ref):
        o_ref[...] = c1_ref[...] * x_ref[...] + c2_ref[...] * n_ref[...]

    return pl.pallas_call(
        body,
        grid=grid,
        in_specs=[
            pl.BlockSpec((1, _B), lambda i: (0, 0)),
            pl.BlockSpec((1, _B), lambda i: (0, 0)),
            pl.BlockSpec((bf, _B), lambda i: (i, 0)),
            pl.BlockSpec((bf, _B), lambda i: (i, 0)),
        ],
        out_specs=pl.BlockSpec((bf, _B), lambda i: (i, 0)),
        out_shape=jax.ShapeDtypeStruct((_D, _B), jnp.float32),
        compiler_params=pltpu.CompilerParams(
            dimension_semantics=("arbitrary",)),
    )(c1r, c2r, xt, nt)


def kernel(x_start, t, noise, sqrt_alphas_cumprod, sqrt_one_minus_alphas_cumprod):
    B, C, H, W = x_start.shape
    c1, c2 = _sc_gather_coeffs(t, sqrt_alphas_cumprod,
                               sqrt_one_minus_alphas_cumprod)
    xt = jnp.transpose(x_start, (1, 2, 3, 0)).reshape(C * H * W, B)
    nt = jnp.transpose(noise, (1, 2, 3, 0)).reshape(C * H * W, B)
    ot = _tc_combine(xt, nt, c1.reshape(1, B), c2.reshape(1, B))
    return jnp.transpose(ot.reshape(C, H, W, B), (3, 0, 1, 2))


# speedup vs baseline: 3.7170x; 1.9777x over previous
import jax
import jax.numpy as jnp
from jax import lax
from jax.experimental import pallas as pl
from jax.experimental.pallas import tpu as pltpu

_B = 256
_D = 4 * 64 * 64
_T_PAD = 1024
_BF = 2048


def _tc_fused(tbl, t2, xt, nt):
    grid = (_D // _BF,)

    def body(tbl_ref, t_ref, x_ref, n_ref, o_ref, c_ref):
        @pl.when(pl.program_id(0) == 0)
        def _():
            tt = t_ref[...]
            rows = lax.broadcasted_iota(jnp.int32, (_T_PAD, _B), 0)
            onehot = jnp.where(rows == tt, 1.0, 0.0)
            c_ref[...] = lax.dot_general(
                tbl_ref[...], onehot,
                dimension_numbers=(((1,), (0,)), ((), ())),
                preferred_element_type=jnp.float32)

        c1 = c_ref[0:1, :]
        c2 = c_ref[1:2, :]
        o_ref[...] = c1 * x_ref[...] + c2 * n_ref[...]

    return pl.pallas_call(
        body,
        grid=grid,
        in_specs=[
            pl.BlockSpec((2, _T_PAD), lambda i: (0, 0)),
            pl.BlockSpec((1, _B), lambda i: (0, 0)),
            pl.BlockSpec((_BF, _B), lambda i: (i, 0)),
            pl.BlockSpec((_BF, _B), lambda i: (i, 0)),
        ],
        out_specs=pl.BlockSpec((_BF, _B), lambda i: (i, 0)),
        out_shape=jax.ShapeDtypeStruct((_D, _B), jnp.float32),
        scratch_shapes=[pltpu.VMEM((2, _B), jnp.float32)],
        compiler_params=pltpu.CompilerParams(
            dimension_semantics=("arbitrary",)),
    )(tbl, t2, xt, nt)


def kernel(x_start, t, noise, sqrt_alphas_cumprod, sqrt_one_minus_alphas_cumprod):
    B, C, H, W = x_start.shape
    tbl = jnp.pad(
        jnp.stack([sqrt_alphas_cumprod, sqrt_one_minus_alphas_cumprod]),
        ((0, 0), (0, _T_PAD - sqrt_alphas_cumprod.shape[0])))
    xt = jnp.transpose(x_start, (1, 2, 3, 0)).reshape(C * H * W, B)
    nt = jnp.transpose(noise, (1, 2, 3, 0)).reshape(C * H * W, B)
    ot = _tc_fused(tbl, t.reshape(1, B), xt, nt)
    return jnp.transpose(ot.reshape(C, H, W, B), (3, 0, 1, 2))
